# baseline (speedup 1.0000x reference)
import jax
import jax.numpy as jnp
from jax import lax
from jax.experimental import pallas as pl
from jax.experimental.pallas import tpu as pltpu
from jax.experimental.pallas import tpu_sc as plsc

EPS = 1e-05
LANES = 16
NUM_CORES = 2
NUM_SUBCORES = 16
NUM_WORKERS = NUM_CORES * NUM_SUBCORES


def _sc_stage(theta_flat, u2, v2, batch, dim, b_per_w):
    n_groups = b_per_w // LANES
    mesh = plsc.VectorSubcoreMesh(core_axis_name="c", subcore_axis_name="s")

    def body(theta_hbm, u_hbm, v_hbm, d2_hbm, prod_hbm,
             iv_u, iv_v, eu, ev, d2_v, prod_v, sem):
        cid = lax.axis_index("c")
        sid = lax.axis_index("s")
        wid = sid * NUM_CORES + cid
        base = wid * b_per_w
        pltpu.sync_copy(u_hbm.at[wid], iv_u)
        pltpu.sync_copy(v_hbm.at[wid], iv_v)
        iota = lax.iota(jnp.int32, LANES)

        def fire(g, c):
            iu = iv_u[pl.ds(g * LANES, LANES)]
            ivv = iv_v[pl.ds(g * LANES, LANES)]
            ou = iu * dim
            ov = ivv * dim
            for l in range(LANES):
                dst = (g * LANES + l) * dim
                pltpu.async_copy(
                    theta_hbm.at[pl.ds(pl.multiple_of(ou[l], 8), dim)],
                    eu.at[pl.ds(dst, dim)], sem)
                pltpu.async_copy(
                    theta_hbm.at[pl.ds(pl.multiple_of(ov[l], 8), dim)],
                    ev.at[pl.ds(dst, dim)], sem)
            return c

        lax.fori_loop(0, n_groups, fire, 0)

        pltpu.make_async_copy(
            theta_hbm.at[pl.ds(0, b_per_w * dim)], eu, sem).wait()
        pltpu.make_async_copy(
            theta_hbm.at[pl.ds(0, b_per_w * dim)], ev, sem).wait()

        def group(g, carry):
            goff = g * LANES
            fbase = (goff + iota) * dim
            nu = jnp.zeros((LANES,), jnp.float32)
            nv = jnp.zeros((LANES,), jnp.float32)
            d2 = jnp.zeros((LANES,), jnp.float32)
            for d in range(dim):
                a = plsc.load_gather(eu, [fbase + d])
                b = plsc.load_gather(ev, [fbase + d])
                nu = nu + a * a
                nv = nv + b * b
                df = a - b
                d2 = d2 + df * df
            one_mu = 1.0 - jnp.minimum(nu, 1.0 - EPS)
            one_mv = 1.0 - jnp.minimum(nv, 1.0 - EPS)
            d2_v[pl.ds(goff, LANES)] = d2
            prod_v[pl.ds(goff, LANES)] = one_mu * one_mv
            return carry

        lax.fori_loop(0, n_groups, group, 0)
        pltpu.sync_copy(d2_v, d2_hbm.at[pl.ds(base, b_per_w)])
        pltpu.sync_copy(prod_v, prod_hbm.at[pl.ds(base, b_per_w)])

    f = pl.kernel(
        body,
        mesh=mesh,
        compiler_params=pltpu.CompilerParams(
            needs_layout_passes=False, use_tc_tiling_on_sc=False),
        out_type=(
            jax.ShapeDtypeStruct((batch,), jnp.float32),
            jax.ShapeDtypeStruct((batch,), jnp.float32),
        ),
        scratch_types=[
            pltpu.VMEM((b_per_w,), jnp.int32),
            pltpu.VMEM((b_per_w,), jnp.int32),
            pltpu.VMEM((b_per_w * dim,), jnp.float32),
            pltpu.VMEM((b_per_w * dim,), jnp.float32),
            pltpu.VMEM((b_per_w,), jnp.float32),
            pltpu.VMEM((b_per_w,), jnp.float32),
            pltpu.SemaphoreType.DMA,
        ],
    )
    return f(theta_flat, u2, v2)


def _tc_body(r_ref, t_ref, d2_ref, prod_ref, o_ref):
    rr = r_ref[0]
    tt = t_ref[0]
    d2 = d2_ref[...]
    pr = prod_ref[...]
    s = 2.0 * jnp.sqrt(d2 + EPS) / pr
    duv = jnp.log(1.0 + s + jnp.sqrt(s * (s + 2.0)))
    o_ref[...] = 1.0 / (jnp.exp((duv - rr) / tt) + 1.0)


def kernel(u, v, theta, r, t):
    batch = u.shape[0]
    dim = theta.shape[1]
    b_per_w = batch // NUM_WORKERS
    theta_flat = theta.reshape(-1)
    u2 = u.reshape(NUM_WORKERS, b_per_w)
    v2 = v.reshape(NUM_WORKERS, b_per_w)
    d2, prod = _sc_stage(theta_flat, u2, v2, batch, dim, b_per_w)

    rows = batch // 128
    out = pl.pallas_call(
        _tc_body,
        out_shape=jax.ShapeDtypeStruct((rows, 128), jnp.float32),
        in_specs=[
            pl.BlockSpec(memory_space=pltpu.SMEM),
            pl.BlockSpec(memory_space=pltpu.SMEM),
            pl.BlockSpec(memory_space=pltpu.VMEM),
            pl.BlockSpec(memory_space=pltpu.VMEM),
        ],
    )(r.reshape(1), t.reshape(1), d2.reshape(rows, 128), prod.reshape(rows, 128))
    return out.reshape(batch)


# speedup vs baseline: 1.4045x; 1.4045x over previous
import jax
import jax.numpy as jnp
from jax import lax
from jax.experimental import pallas as pl
from jax.experimental.pallas import tpu as pltpu
from jax.experimental.pallas import tpu_sc as plsc

EPS = 1e-05
LANES = 16
NUM_CORES = 2
NUM_SUBCORES = 16
NUM_WORKERS = NUM_CORES * NUM_SUBCORES
BLK = 8
PASS_ROWS = 32


def _sc_stage(theta, u2, v2, batch, dim, b_per_w):
    n_pass = b_per_w // PASS_ROWS
    n_groups = PASS_ROWS // LANES
    mesh = plsc.VectorSubcoreMesh(core_axis_name="c", subcore_axis_name="s")

    def body(theta_hbm, u_hbm, v_hbm, d2_hbm, prod_hbm,
             iv_u, iv_v, stage_u, stage_v, d2_v, prod_v, sem):
        cid = lax.axis_index("c")
        sid = lax.axis_index("s")
        wid = sid * NUM_CORES + cid
        base = wid * b_per_w
        pltpu.sync_copy(u_hbm.at[wid], iv_u)
        pltpu.sync_copy(v_hbm.at[wid], iv_v)
        iota = lax.iota(jnp.int32, LANES)

        def do_pass(p, carry):
            poff = p * PASS_ROWS

            def fire(gg, c):
                iu = iv_u[pl.ds(poff + gg * LANES, LANES)]
                ivv = iv_v[pl.ds(poff + gg * LANES, LANES)]
                bu_v = lax.shift_right_logical(iu, 3) * BLK
                bv_v = lax.shift_right_logical(ivv, 3) * BLK
                for l in range(LANES):
                    bu = bu_v[l]
                    bv = bv_v[l]
                    dst = (gg * LANES + l) * BLK
                    pltpu.async_copy(
                        theta_hbm.at[pl.ds(pl.multiple_of(bu, BLK), BLK)],
                        stage_u.at[pl.ds(dst, BLK)], sem)
                    pltpu.async_copy(
                        theta_hbm.at[pl.ds(pl.multiple_of(bv, BLK), BLK)],
                        stage_v.at[pl.ds(dst, BLK)], sem)
                return c

            lax.fori_loop(0, PASS_ROWS // LANES, fire, 0)

            pltpu.make_async_copy(
                theta_hbm.at[pl.ds(0, PASS_ROWS * BLK)], stage_u, sem).wait()
            pltpu.make_async_copy(
                theta_hbm.at[pl.ds(0, PASS_ROWS * BLK)], stage_v, sem).wait()

            for g in range(n_groups):
                goff = poff + g * LANES
                iu = iv_u[pl.ds(goff, LANES)]
                ivv = iv_v[pl.ds(goff, LANES)]
                srow_u = (g * LANES + iota) * BLK + (iu & (BLK - 1))
                srow_v = (g * LANES + iota) * BLK + (ivv & (BLK - 1))
                nu = jnp.zeros((LANES,), jnp.float32)
                nv = jnp.zeros((LANES,), jnp.float32)
                d2 = jnp.zeros((LANES,), jnp.float32)
                for d in range(dim):
                    col = jnp.full((LANES,), d, jnp.int32)
                    a = plsc.load_gather(stage_u, [srow_u, col])
                    b = plsc.load_gather(stage_v, [srow_v, col])
                    nu = nu + a * a
                    nv = nv + b * b
                    df = a - b
                    d2 = d2 + df * df
                one_mu = 1.0 - jnp.minimum(nu, 1.0 - EPS)
                one_mv = 1.0 - jnp.minimum(nv, 1.0 - EPS)
                d2_v[pl.ds(goff, LANES)] = d2
                prod_v[pl.ds(goff, LANES)] = one_mu * one_mv
            return carry

        lax.fori_loop(0, n_pass, do_pass, 0)
        pltpu.sync_copy(d2_v, d2_hbm.at[pl.ds(base, b_per_w)])
        pltpu.sync_copy(prod_v, prod_hbm.at[pl.ds(base, b_per_w)])

    f = pl.kernel(
        body,
        mesh=mesh,
        compiler_params=pltpu.CompilerParams(
            needs_layout_passes=False, use_tc_tiling_on_sc=True),
        out_type=(
            jax.ShapeDtypeStruct((batch,), jnp.float32),
            jax.ShapeDtypeStruct((batch,), jnp.float32),
        ),
        scratch_types=[
            pltpu.VMEM((b_per_w,), jnp.int32),
            pltpu.VMEM((b_per_w,), jnp.int32),
            pltpu.VMEM((PASS_ROWS * BLK, dim), jnp.float32),
            pltpu.VMEM((PASS_ROWS * BLK, dim), jnp.float32),
            pltpu.VMEM((b_per_w,), jnp.float32),
            pltpu.VMEM((b_per_w,), jnp.float32),
            pltpu.SemaphoreType.DMA,
        ],
    )
    return f(theta, u2, v2)


def _tc_body(r_ref, t_ref, d2_ref, prod_ref, o_ref):
    rr = r_ref[0]
    tt = t_ref[0]
    d2 = d2_ref[...]
    pr = prod_ref[...]
    s = 2.0 * jnp.sqrt(d2 + EPS) / pr
    duv = jnp.log(1.0 + s + jnp.sqrt(s * (s + 2.0)))
    o_ref[...] = 1.0 / (jnp.exp((duv - rr) / tt) + 1.0)


def kernel(u, v, theta, r, t):
    batch = u.shape[0]
    dim = theta.shape[1]
    b_per_w = batch // NUM_WORKERS
    u2 = u.reshape(NUM_WORKERS, b_per_w)
    v2 = v.reshape(NUM_WORKERS, b_per_w)
    d2, prod = _sc_stage(theta, u2, v2, batch, dim, b_per_w)

    rows = batch // 128
    out = pl.pallas_call(
        _tc_body,
        out_shape=jax.ShapeDtypeStruct((rows, 128), jnp.float32),
        in_specs=[
            pl.BlockSpec(memory_space=pltpu.SMEM),
            pl.BlockSpec(memory_space=pltpu.SMEM),
            pl.BlockSpec(memory_space=pltpu.VMEM),
            pl.BlockSpec(memory_space=pltpu.VMEM),
        ],
    )(r.reshape(1), t.reshape(1), d2.reshape(rows, 128), prod.reshape(rows, 128))
    return out.reshape(batch)


# speedup vs baseline: 1.4509x; 1.0331x over previous
import jax
import jax.numpy as jnp
from jax import lax
from jax.experimental import pallas as pl
from jax.experimental.pallas import tpu as pltpu
from jax.experimental.pallas import tpu_sc as plsc

EPS = 1e-05
LANES = 16
NUM_CORES = 2
NUM_SUBCORES = 16
NUM_WORKERS = NUM_CORES * NUM_SUBCORES
BLK = 8
PASS_ROWS = 16


def _sc_stage(theta, u2, v2, batch, dim, b_per_w):
    n_pass = b_per_w // PASS_ROWS
    mesh = plsc.VectorSubcoreMesh(core_axis_name="c", subcore_axis_name="s")

    def body(theta_hbm, u_hbm, v_hbm, d2_hbm, prod_hbm,
             iv_u, iv_v, su0, sv0, su1, sv1, d2_v, prod_v, sem0, sem1):
        cid = lax.axis_index("c")
        sid = lax.axis_index("s")
        wid = sid * NUM_CORES + cid
        base = wid * b_per_w
        pltpu.sync_copy(u_hbm.at[wid], iv_u)
        pltpu.sync_copy(v_hbm.at[wid], iv_v)
        iota = lax.iota(jnp.int32, LANES)

        def fire(p, stage_u, stage_v, sem):
            iu = iv_u[pl.ds(p * PASS_ROWS, LANES)]
            ivv = iv_v[pl.ds(p * PASS_ROWS, LANES)]
            bu_v = lax.shift_right_logical(iu, 3) * BLK
            bv_v = lax.shift_right_logical(ivv, 3) * BLK
            for l in range(LANES):
                dst = l * BLK
                pltpu.async_copy(
                    theta_hbm.at[pl.ds(pl.multiple_of(bu_v[l], BLK), BLK)],
                    stage_u.at[pl.ds(dst, BLK)], sem)
                pltpu.async_copy(
                    theta_hbm.at[pl.ds(pl.multiple_of(bv_v[l], BLK), BLK)],
                    stage_v.at[pl.ds(dst, BLK)], sem)

        def wait(stage_u, stage_v, sem):
            pltpu.make_async_copy(
                theta_hbm.at[pl.ds(0, PASS_ROWS * BLK)], stage_u, sem).wait()
            pltpu.make_async_copy(
                theta_hbm.at[pl.ds(0, PASS_ROWS * BLK)], stage_v, sem).wait()

        def compute(p, stage_u, stage_v):
            goff = p * PASS_ROWS
            iu = iv_u[pl.ds(goff, LANES)]
            ivv = iv_v[pl.ds(goff, LANES)]
            srow_u = iota * BLK + (iu & (BLK - 1))
            srow_v = iota * BLK + (ivv & (BLK - 1))
            nu = jnp.zeros((LANES,), jnp.float32)
            nv = jnp.zeros((LANES,), jnp.float32)
            d2 = jnp.zeros((LANES,), jnp.float32)
            for d in range(dim):
                col = jnp.full((LANES,), d, jnp.int32)
                a = plsc.load_gather(stage_u, [srow_u, col])
                b = plsc.load_gather(stage_v, [srow_v, col])
                nu = nu + a * a
                nv = nv + b * b
                df = a - b
                d2 = d2 + df * df
            one_mu = 1.0 - jnp.minimum(nu, 1.0 - EPS)
            one_mv = 1.0 - jnp.minimum(nv, 1.0 - EPS)
            d2_v[pl.ds(goff, LANES)] = d2
            prod_v[pl.ds(goff, LANES)] = one_mu * one_mv

        fire(0, su0, sv0, sem0)

        def pair(k, carry):
            fire(2 * k + 1, su1, sv1, sem1)
            wait(su0, sv0, sem0)
            compute(2 * k, su0, sv0)

            @pl.when(k < n_pass // 2 - 1)
            def _():
                fire(2 * k + 2, su0, sv0, sem0)

            wait(su1, sv1, sem1)
            compute(2 * k + 1, su1, sv1)
            return carry

        lax.fori_loop(0, n_pass // 2, pair, 0)
        pltpu.sync_copy(d2_v, d2_hbm.at[pl.ds(base, b_per_w)])
        pltpu.sync_copy(prod_v, prod_hbm.at[pl.ds(base, b_per_w)])

    f = pl.kernel(
        body,
        mesh=mesh,
        compiler_params=pltpu.CompilerParams(
            needs_layout_passes=False, use_tc_tiling_on_sc=True),
        out_type=(
            jax.ShapeDtypeStruct((batch,), jnp.float32),
            jax.ShapeDtypeStruct((batch,), jnp.float32),
        ),
        scratch_types=[
            pltpu.VMEM((b_per_w,), jnp.int32),
            pltpu.VMEM((b_per_w,), jnp.int32),
            pltpu.VMEM((PASS_ROWS * BLK, dim), jnp.float32),
            pltpu.VMEM((PASS_ROWS * BLK, dim), jnp.float32),
            pltpu.VMEM((PASS_ROWS * BLK, dim), jnp.float32),
            pltpu.VMEM((PASS_ROWS * BLK, dim), jnp.float32),
            pltpu.VMEM((b_per_w,), jnp.float32),
            pltpu.VMEM((b_per_w,), jnp.float32),
            pltpu.SemaphoreType.DMA,
            pltpu.SemaphoreType.DMA,
        ],
    )
    return f(theta, u2, v2)


def _tc_body(r_ref, t_ref, d2_ref, prod_ref, o_ref):
    rr = r_ref[0]
    tt = t_ref[0]
    d2 = d2_ref[...]
    pr = prod_ref[...]
    s = 2.0 * jnp.sqrt(d2 + EPS) / pr
    duv = jnp.log(1.0 + s + jnp.sqrt(s * (s + 2.0)))
    o_ref[...] = 1.0 / (jnp.exp((duv - rr) / tt) + 1.0)


def kernel(u, v, theta, r, t):
    batch = u.shape[0]
    dim = theta.shape[1]
    b_per_w = batch // NUM_WORKERS
    u2 = u.reshape(NUM_WORKERS, b_per_w)
    v2 = v.reshape(NUM_WORKERS, b_per_w)
    d2, prod = _sc_stage(theta, u2, v2, batch, dim, b_per_w)

    rows = batch // 128
    out = pl.pallas_call(
        _tc_body,
        out_shape=jax.ShapeDtypeStruct((rows, 128), jnp.float32),
        in_specs=[
            pl.BlockSpec(memory_space=pltpu.SMEM),
            pl.BlockSpec(memory_space=pltpu.SMEM),
            pl.BlockSpec(memory_space=pltpu.VMEM),
            pl.BlockSpec(memory_space=pltpu.VMEM),
        ],
    )(r.reshape(1), t.reshape(1), d2.reshape(rows, 128), prod.reshape(rows, 128))
    return out.reshape(batch)
